# baseline (speedup 1.0000x reference)
import jax
import jax.numpy as jnp
from jax import lax
from jax.experimental import pallas as pl
from jax.experimental.pallas import tpu as pltpu
from jax.experimental.pallas import tpu_sc as plsc

N_ROWS = 1024
VOCAB = 100000
NEG = 64
NC, NS = 2, 16
NW = NC * NS
CHUNK = N_ROWS * NEG // NW
IDXW = 128
JCH = CHUNK // IDXW
POSW = N_ROWS // NW


def _sc_gather(flat, nidx, pidx):
    def body(flat_hbm, nidx_hbm, pidx_hbm, neg_out, pos_out,
             nidx_v, pidx_v, nval_v, pval_v, sem):
        wid = lax.axis_index("s") * NC + lax.axis_index("c")
        pltpu.sync_copy(nidx_hbm.at[wid], nidx_v)
        pltpu.sync_copy(pidx_hbm.at[wid], pidx_v)
        copies = [pltpu.async_copy(flat_hbm.at[nidx_v.at[j]], nval_v.at[j], sem)
                  for j in range(JCH)]
        copies.append(pltpu.async_copy(flat_hbm.at[pidx_v], pval_v, sem))
        for c in copies:
            c.wait()
        pltpu.sync_copy(nval_v, neg_out.at[wid])
        pltpu.sync_copy(pval_v, pos_out.at[wid])

    fn = pl.kernel(
        body,
        out_type=(
            jax.ShapeDtypeStruct((NW, JCH, IDXW), jnp.float32),
            jax.ShapeDtypeStruct((NW, POSW), jnp.float32),
        ),
        mesh=plsc.VectorSubcoreMesh(core_axis_name="c", subcore_axis_name="s"),
        scratch_types=[
            pltpu.VMEM((JCH, IDXW), jnp.int32),
            pltpu.VMEM((POSW,), jnp.int32),
            pltpu.VMEM((JCH, IDXW), jnp.float32),
            pltpu.VMEM((POSW,), jnp.float32),
            pltpu.SemaphoreType.DMA,
        ],
    )
    return fn(flat, nidx, pidx)


def _tc_loss(neg, pos):
    def body(neg_ref, pos_ref, out_ref):
        z = pos_ref[...] - neg_ref[...]
        sp = jnp.maximum(-z, 0.0) + jnp.log1p(jnp.exp(-jnp.abs(z)))
        out_ref[0, 0] = jnp.sum(sp) * (1.0 / (N_ROWS * NEG))

    return pl.pallas_call(
        body,
        out_shape=jax.ShapeDtypeStruct((1, 1), jnp.float32),
        out_specs=pl.BlockSpec(memory_space=pltpu.SMEM),
    )(neg, pos)


def kernel(input, target, neg_action):
    row_off = jnp.arange(N_ROWS, dtype=jnp.int32) * VOCAB
    nidx = (row_off[:, None] + neg_action).reshape(NW, JCH, IDXW)
    pidx = (row_off + target).reshape(NW, POSW)
    flat = input.reshape(-1)
    neg_g, pos_g = _sc_gather(flat, nidx, pidx)
    neg = neg_g.reshape(N_ROWS, NEG)
    pos = pos_g.reshape(N_ROWS, 1)
    return _tc_loss(neg, pos)[0, 0]


# speedup vs baseline: 28.4464x; 28.4464x over previous
import jax
import jax.numpy as jnp
from jax import lax
from jax.experimental import pallas as pl
from jax.experimental.pallas import tpu as pltpu
from jax.experimental.pallas import tpu_sc as plsc

N_ROWS = 1024
VOCAB = 100000
NEG = 64
NC, NS = 2, 16
NW = NC * NS
CHUNK = N_ROWS * NEG // NW
IDXW = 128
JCH = CHUNK // IDXW
POSW = N_ROWS // NW


def _sc_gather(flat, nidx, pidx):
    def body(flat_hbm, nidx_hbm, pidx_hbm, neg_out, pos_out,
             nidx_v, pidx_v, nval_v, pval_v, sem):
        wid = lax.axis_index("s") * NC + lax.axis_index("c")
        pltpu.sync_copy(nidx_hbm.at[wid], nidx_v)
        pltpu.sync_copy(pidx_hbm.at[wid], pidx_v)
        copies = [pltpu.async_copy(flat_hbm.at[nidx_v.at[j]], nval_v.at[j], sem)
                  for j in range(JCH)]
        copies.append(pltpu.async_copy(flat_hbm.at[pidx_v], pval_v, sem))
        for c in copies:
            c.wait()
        pltpu.sync_copy(nval_v, neg_out.at[wid])
        pltpu.sync_copy(pval_v, pos_out.at[wid])

    fn = pl.kernel(
        body,
        out_type=(
            jax.ShapeDtypeStruct((NW, JCH, IDXW), jnp.float32),
            jax.ShapeDtypeStruct((NW, POSW), jnp.float32),
        ),
        mesh=plsc.VectorSubcoreMesh(core_axis_name="c", subcore_axis_name="s"),
        scratch_types=[
            pltpu.VMEM((JCH, IDXW), jnp.int32),
            pltpu.VMEM((POSW,), jnp.int32),
            pltpu.VMEM((JCH, IDXW), jnp.float32),
            pltpu.VMEM((POSW,), jnp.float32),
            pltpu.SemaphoreType.DMA,
        ],
    )
    return fn(flat, nidx, pidx)


def _tc_loss(neg, pos):
    def body(neg_ref, pos_ref, out_ref):
        z = pos_ref[...] - neg_ref[...]
        sp = jnp.maximum(-z, 0.0) + jnp.log1p(jnp.exp(-jnp.abs(z)))
        out_ref[0, 0] = jnp.sum(sp) * (1.0 / (N_ROWS * NEG))

    return pl.pallas_call(
        body,
        out_shape=jax.ShapeDtypeStruct((1, 1), jnp.float32),
        out_specs=pl.BlockSpec(memory_space=pltpu.SMEM),
    )(neg, pos)


def _phys_idx(row, col):
    return ((col >> 3) * 8 + (row >> 7)) * 1024 + (col & 7) * 128 + (row & 127)


def kernel(input, target, neg_action):
    flat = input.reshape(8, 128, VOCAB // 8, 8).transpose(2, 0, 3, 1).reshape(-1)
    row = jnp.arange(N_ROWS, dtype=jnp.int32)
    nidx = _phys_idx(row[:, None], neg_action).reshape(NW, JCH, IDXW)
    pidx = _phys_idx(row, target).reshape(NW, POSW)
    neg_g, pos_g = _sc_gather(flat, nidx, pidx)
    neg = neg_g.reshape(N_ROWS, NEG)
    pos = pos_g.reshape(N_ROWS, 1)
    return _tc_loss(neg, pos)[0, 0]
